# baseline (speedup 1.0000x reference)
import functools

import jax
import jax.numpy as jnp
from jax import lax
from jax.experimental import pallas as pl
from jax.experimental.pallas import tpu as pltpu
from jax.experimental.pallas import tpu_sc as plsc

DIM = 128
FIELDS = 26
NW = 32
CHUNK = 128


def _sc_body(z_hbm, noise_hbm, out_hbm, zbuf, buf):
    wid = lax.axis_index("s") * 2 + lax.axis_index("c")
    rows_total = z_hbm.shape[0]
    rows_per_w = rows_total // NW
    base = wid * rows_per_w
    lane = lax.iota(jnp.int32, 16)

    def chunk_step(c, carry):
        r0 = base + c * CHUNK
        pltpu.sync_copy(noise_hbm.at[pl.ds(r0 * DIM, CHUNK * DIM)], buf)
        pltpu.sync_copy(z_hbm.at[pl.ds(r0, CHUNK)], zbuf)

        def rowblock(i, carry2):
            z16 = zbuf[pl.ds(i * 16, 16)]
            rbase = i * 16 * DIM
            for t in range(16):
                bz = z16.at[jnp.full((16,), t, jnp.int32)].get(
                    mode="promise_in_bounds")
                for g in range(8):
                    sl = pl.ds(rbase + t * DIM + g * 16, 16)
                    v = buf[sl]
                    buf[sl] = jnp.where(lane + 16 * g == bz, v + 1.0, v)
            return carry2

        lax.fori_loop(0, CHUNK // 16, rowblock, 0)
        pltpu.sync_copy(buf, out_hbm.at[pl.ds(r0 * DIM, CHUNK * DIM)])
        return carry

    lax.fori_loop(0, rows_per_w // CHUNK, chunk_step, 0)


def kernel(z, noise):
    batch = z.shape[0]
    rows = batch * FIELDS
    z_flat = z.T.reshape(rows)
    noise1d = jnp.transpose(noise, (1, 0, 2)).reshape(rows * DIM)

    mesh = plsc.VectorSubcoreMesh(core_axis_name="c", subcore_axis_name="s")
    sc_call = functools.partial(
        pl.kernel,
        mesh=mesh,
        out_type=jax.ShapeDtypeStruct((rows * DIM,), jnp.float32),
        scratch_types=[
            pltpu.VMEM((CHUNK,), jnp.int32),
            pltpu.VMEM((CHUNK * DIM,), jnp.float32),
        ],
    )

    @sc_call
    def run(z_hbm, noise_hbm, out_hbm, zbuf, buf):
        _sc_body(z_hbm, noise_hbm, out_hbm, zbuf, buf)

    out1d = run(z_flat, noise1d)
    out = jnp.transpose(out1d.reshape(FIELDS, batch, DIM), (1, 0, 2))
    return (out, 0)


# speedup vs baseline: 1.1874x; 1.1874x over previous
import functools

import jax
import jax.numpy as jnp
from jax import lax
from jax.experimental import pallas as pl
from jax.experimental.pallas import tpu as pltpu
from jax.experimental.pallas import tpu_sc as plsc

DIM = 128
FIELDS = 26
F_TC = 19
F_SC = FIELDS - F_TC
NW = 32
CHUNK = 128
BATCH_BLK = 512


def _tc_kernel(z_ref, noise_ref, out_ref):
    z = z_ref[...][:F_TC]
    iota = lax.broadcasted_iota(jnp.int32, (F_TC, BATCH_BLK, DIM), 2)
    mask = (z[:, :, None] == iota).astype(jnp.float32)
    out_ref[...] = noise_ref[...] + mask


def _sc_body(z_hbm, noise_hbm, out_hbm, zbuf, buf, row0):
    wid = lax.axis_index("s") * 2 + lax.axis_index("c")
    rows_here = F_SC * 16384
    rows_per_w = rows_here // NW
    base = row0 + wid * rows_per_w
    lane = lax.iota(jnp.int32, 16)

    def chunk_step(c, carry):
        r0 = base + c * CHUNK
        pltpu.sync_copy(noise_hbm.at[pl.ds(r0 * DIM, CHUNK * DIM)], buf)
        pltpu.sync_copy(z_hbm.at[pl.ds(r0, CHUNK)], zbuf)

        def rowblock(i, carry2):
            z16 = zbuf[pl.ds(i * 16, 16)]
            rbase = i * 16 * DIM
            for t in range(16):
                bz = z16.at[jnp.full((16,), t, jnp.int32)].get(
                    mode="promise_in_bounds")
                for g in range(8):
                    sl = pl.ds(rbase + t * DIM + g * 16, 16)
                    v = buf[sl]
                    buf[sl] = jnp.where(lane + 16 * g == bz, v + 1.0, v)
            return carry2

        lax.fori_loop(0, CHUNK // 16, rowblock, 0)
        pltpu.sync_copy(buf, out_hbm.at[pl.ds((r0 - row0) * DIM, CHUNK * DIM)])
        return carry

    lax.fori_loop(0, rows_per_w // CHUNK, chunk_step, 0)


def kernel(z, noise):
    batch = z.shape[0]
    rows = batch * FIELDS
    split_row = F_TC * batch
    z_t = z.T
    noise_t = jnp.transpose(noise, (1, 0, 2))
    z_flat = z_t.reshape(rows)
    noise1d = noise_t.reshape(rows * DIM)

    mesh = plsc.VectorSubcoreMesh(core_axis_name="c", subcore_axis_name="s")
    sc_call = functools.partial(
        pl.kernel,
        mesh=mesh,
        out_type=jax.ShapeDtypeStruct((F_SC * batch * DIM,), jnp.float32),
        scratch_types=[
            pltpu.VMEM((CHUNK,), jnp.int32),
            pltpu.VMEM((CHUNK * DIM,), jnp.float32),
        ],
    )

    @sc_call
    def run_sc(z_hbm, noise_hbm, out_hbm, zbuf, buf):
        _sc_body(z_hbm, noise_hbm, out_hbm, zbuf, buf, split_row)

    out_sc = run_sc(z_flat, noise1d)

    out_tc = pl.pallas_call(
        _tc_kernel,
        grid=(batch // BATCH_BLK,),
        in_specs=[
            pl.BlockSpec((FIELDS, BATCH_BLK), lambda i: (0, i)),
            pl.BlockSpec((F_TC, BATCH_BLK, DIM), lambda i: (0, i, 0)),
        ],
        out_specs=pl.BlockSpec((F_TC, BATCH_BLK, DIM), lambda i: (0, i, 0)),
        out_shape=jax.ShapeDtypeStruct((F_TC, batch, DIM), jnp.float32),
    )(z_t, noise_t)

    out1d = jnp.concatenate([out_tc.reshape(split_row * DIM), out_sc])
    out = jnp.transpose(out1d.reshape(FIELDS, batch, DIM), (1, 0, 2))
    return (out, 0)


# speedup vs baseline: 2.0504x; 1.7268x over previous
import functools

import jax
import jax.numpy as jnp
from jax import lax
from jax.experimental import pallas as pl
from jax.experimental.pallas import tpu as pltpu
from jax.experimental.pallas import tpu_sc as plsc

DIM = 128
FIELDS = 26
F_TC = 24
F_SC = FIELDS - F_TC
NW = 32
CHUNK = 128
BATCH_BLK = 512


def _tc_kernel(z_ref, noise_ref, prev_ref, out_ref):
    del prev_ref
    z = z_ref[...]
    iota = lax.broadcasted_iota(jnp.int32, (F_TC, BATCH_BLK, DIM), 2)
    mask = (z[:, :, None] == iota).astype(jnp.float32)
    out_ref[...] = noise_ref[...] + mask


def _sc_body(z_hbm, noise_hbm, out_hbm, zbuf, buf, row0):
    wid = lax.axis_index("s") * 2 + lax.axis_index("c")
    rows_here = F_SC * 16384
    rows_per_w = rows_here // NW
    base = row0 + wid * rows_per_w
    lane = lax.iota(jnp.int32, 16)

    def chunk_step(c, carry):
        r0 = base + c * CHUNK
        pltpu.sync_copy(noise_hbm.at[pl.ds(r0 * DIM, CHUNK * DIM)], buf)
        pltpu.sync_copy(z_hbm.at[pl.ds(r0, CHUNK)], zbuf)

        def rowblock(i, carry2):
            z16 = zbuf[pl.ds(i * 16, 16)]
            rbase = i * 16 * DIM
            for t in range(16):
                bz = z16.at[jnp.full((16,), t, jnp.int32)].get(
                    mode="promise_in_bounds")
                for g in range(8):
                    sl = pl.ds(rbase + t * DIM + g * 16, 16)
                    v = buf[sl]
                    buf[sl] = jnp.where(lane + 16 * g == bz, v + 1.0, v)
            return carry2

        lax.fori_loop(0, CHUNK // 16, rowblock, 0)
        pltpu.sync_copy(buf, out_hbm.at[pl.ds(r0 * DIM, CHUNK * DIM)])
        return carry

    lax.fori_loop(0, rows_per_w // CHUNK, chunk_step, 0)


def kernel(z, noise):
    batch = z.shape[0]
    rows = batch * FIELDS
    split_row = F_TC * batch
    z_t = z.T
    noise_t = jnp.transpose(noise, (1, 0, 2))
    z_flat = z_t.reshape(rows)
    noise1d = noise_t.reshape(rows * DIM)

    mesh = plsc.VectorSubcoreMesh(core_axis_name="c", subcore_axis_name="s")
    sc_call = functools.partial(
        pl.kernel,
        mesh=mesh,
        out_type=jax.ShapeDtypeStruct((rows * DIM,), jnp.float32),
        scratch_types=[
            pltpu.VMEM((CHUNK,), jnp.int32),
            pltpu.VMEM((CHUNK * DIM,), jnp.float32),
        ],
    )

    @sc_call
    def run_sc(z_hbm, noise_hbm, out_hbm, zbuf, buf):
        _sc_body(z_hbm, noise_hbm, out_hbm, zbuf, buf, split_row)

    out_sc = run_sc(z_flat, noise1d)
    out_sc3 = out_sc.reshape(FIELDS, batch, DIM)

    out_t = pl.pallas_call(
        _tc_kernel,
        grid=(batch // BATCH_BLK,),
        in_specs=[
            pl.BlockSpec((F_TC, BATCH_BLK), lambda i: (0, i)),
            pl.BlockSpec((F_TC, BATCH_BLK, DIM), lambda i: (0, i, 0)),
            pl.BlockSpec(memory_space=pl.ANY),
        ],
        out_specs=pl.BlockSpec((F_TC, BATCH_BLK, DIM), lambda i: (0, i, 0)),
        out_shape=jax.ShapeDtypeStruct((FIELDS, batch, DIM), jnp.float32),
        input_output_aliases={2: 0},
    )(z_t, noise_t, out_sc3)

    out = jnp.transpose(out_t, (1, 0, 2))
    return (out, 0)


# speedup vs baseline: 2.5954x; 1.2658x over previous
import jax
import jax.numpy as jnp
from jax import lax
from jax.experimental import pallas as pl

DIM = 128
FIELDS = 26
BATCH_BLK = 1024


def _onehot_add_kernel(z_ref, noise_ref, out_ref):
    z = z_ref[...]
    iota = lax.broadcasted_iota(jnp.int32, (FIELDS, BATCH_BLK, DIM), 2)
    mask = (z[:, :, None] == iota).astype(jnp.float32)
    out_ref[...] = noise_ref[...] + mask


def kernel(z, noise):
    batch = z.shape[0]
    z_t = z.T
    noise_t = jnp.transpose(noise, (1, 0, 2))
    grid = (batch // BATCH_BLK,)
    out_t = pl.pallas_call(
        _onehot_add_kernel,
        grid=grid,
        in_specs=[
            pl.BlockSpec((FIELDS, BATCH_BLK), lambda i: (0, i)),
            pl.BlockSpec((FIELDS, BATCH_BLK, DIM), lambda i: (0, i, 0)),
        ],
        out_specs=pl.BlockSpec((FIELDS, BATCH_BLK, DIM), lambda i: (0, i, 0)),
        out_shape=jax.ShapeDtypeStruct((FIELDS, batch, DIM), jnp.float32),
    )(z_t, noise_t)
    out = jnp.transpose(out_t, (1, 0, 2))
    return (out, 0)
